# baseline (speedup 1.0000x reference)
import functools

import jax
import jax.numpy as jnp
from jax.experimental import pallas as pl
from jax.experimental.pallas import tpu as pltpu

_WEIGHT = 0.6
_MCHUNK = 2048


def _chamfer_body(a_ref, bt_ref, out_ref, rmin_ref, acc_ref, *, rscale, cscale):
    b = pl.program_id(0)
    mi = pl.program_id(1)
    nb = pl.num_programs(0)
    nm = pl.num_programs(1)
    N = a_ref.shape[1]
    f32 = jnp.float32

    a = a_ref[0]
    bt = bt_ref[0].T

    a6 = jnp.concatenate([a, jnp.ones((N, 3), f32)], axis=1)
    bb = jnp.sum(bt * bt, axis=0, keepdims=True)
    hi = bb.astype(jnp.bfloat16).astype(f32)
    r1 = bb - hi
    mid = r1.astype(jnp.bfloat16).astype(f32)
    lo = r1 - mid
    b6 = jnp.concatenate([-2.0 * bt, hi, mid, lo], axis=0)

    ab2 = jax.lax.dot_general(
        a6, b6, (((1,), (0,)), ((), ())),
        preferred_element_type=f32)
    aa = jnp.sum(a * a, axis=1, keepdims=True)

    cmin = jnp.maximum(jnp.min(ab2 + aa, axis=0), 0.0)
    csum = jnp.sum(cmin)

    rmin_chunk = jnp.min(ab2, axis=1, keepdims=True)

    @pl.when(mi == 0)
    def _():
        rmin_ref[...] = rmin_chunk

    @pl.when(mi != 0)
    def _():
        rmin_ref[...] = jnp.minimum(rmin_ref[...], rmin_chunk)

    @pl.when(jnp.logical_and(b == 0, mi == 0))
    def _():
        acc_ref[0] = 0.0

    acc_ref[0] += csum * cscale

    @pl.when(mi == nm - 1)
    def _():
        rsum = jnp.sum(jnp.maximum(rmin_ref[...] + aa, 0.0))
        acc_ref[0] += rsum * rscale

    @pl.when(jnp.logical_and(b == nb - 1, mi == nm - 1))
    def _():
        out_ref[0, 0] = acc_ref[0]


def kernel(xyz1, xyz2):
    B, N, _ = xyz1.shape
    M = xyz2.shape[1]
    f32 = jnp.float32

    nm = M // _MCHUNK
    rscale = 0.5 * _WEIGHT / (B * N)
    cscale = 0.5 * _WEIGHT / (B * M)

    out = pl.pallas_call(
        functools.partial(_chamfer_body, rscale=rscale, cscale=cscale),
        grid=(B, nm),
        in_specs=[
            pl.BlockSpec((1, N, 3), lambda b, mi: (b, 0, 0)),
            pl.BlockSpec((1, _MCHUNK, 3), lambda b, mi: (b, mi, 0)),
        ],
        out_specs=pl.BlockSpec(memory_space=pltpu.SMEM),
        out_shape=jax.ShapeDtypeStruct((1, 1), f32),
        scratch_shapes=[
            pltpu.VMEM((N, 1), f32),
            pltpu.SMEM((1,), f32),
        ],
    )(xyz1, xyz2)
    return out[0, 0]


# speedup vs baseline: 1.0444x; 1.0444x over previous
import functools

import jax
import jax.numpy as jnp
from jax.experimental import pallas as pl
from jax.experimental.pallas import tpu as pltpu

_WEIGHT = 0.6
_NCHUNK = 2048


def _chamfer_body(a_ref, bt_ref, out_ref, cmin_ref, acc_ref, *, rscale, cscale):
    b = pl.program_id(0)
    ni = pl.program_id(1)
    nb = pl.num_programs(0)
    nn = pl.num_programs(1)
    NC = a_ref.shape[1]
    M = bt_ref.shape[2]
    f32 = jnp.float32

    a = a_ref[0]
    bt = bt_ref[0]

    a6 = jnp.concatenate([a, jnp.ones((NC, 3), f32)], axis=1)
    bb = jnp.sum(bt * bt, axis=0, keepdims=True)
    hi = bb.astype(jnp.bfloat16).astype(f32)
    r1 = bb - hi
    mid = r1.astype(jnp.bfloat16).astype(f32)
    lo = r1 - mid
    b6 = jnp.concatenate([-2.0 * bt, hi, mid, lo], axis=0)

    ab2 = jax.lax.dot_general(
        a6, b6, (((1,), (0,)), ((), ())),
        preferred_element_type=f32)
    aa = jnp.sum(a * a, axis=1, keepdims=True)

    rmin = jnp.maximum(jnp.min(ab2, axis=1, keepdims=True) + aa, 0.0)
    rsum = jnp.sum(rmin)

    cmin_chunk = jnp.min(ab2 + aa, axis=0, keepdims=True)

    @pl.when(ni == 0)
    def _():
        cmin_ref[...] = cmin_chunk

    @pl.when(ni != 0)
    def _():
        cmin_ref[...] = jnp.minimum(cmin_ref[...], cmin_chunk)

    @pl.when(jnp.logical_and(b == 0, ni == 0))
    def _():
        acc_ref[0] = 0.0

    acc_ref[0] += rsum * rscale

    @pl.when(ni == nn - 1)
    def _():
        csum = jnp.sum(jnp.maximum(cmin_ref[...], 0.0))
        acc_ref[0] += csum * cscale

    @pl.when(jnp.logical_and(b == nb - 1, ni == nn - 1))
    def _():
        out_ref[0, 0] = acc_ref[0]


def kernel(xyz1, xyz2):
    B, N, _ = xyz1.shape
    M = xyz2.shape[1]
    f32 = jnp.float32

    bt = jnp.swapaxes(xyz2, 1, 2)

    nn = N // _NCHUNK
    rscale = 0.5 * _WEIGHT / (B * N)
    cscale = 0.5 * _WEIGHT / (B * M)

    out = pl.pallas_call(
        functools.partial(_chamfer_body, rscale=rscale, cscale=cscale),
        grid=(B, nn),
        in_specs=[
            pl.BlockSpec((1, _NCHUNK, 3), lambda b, ni: (b, ni, 0)),
            pl.BlockSpec((1, 3, M), lambda b, ni: (b, 0, 0)),
        ],
        out_specs=pl.BlockSpec(memory_space=pltpu.SMEM),
        out_shape=jax.ShapeDtypeStruct((1, 1), f32),
        scratch_shapes=[
            pltpu.VMEM((1, M), f32),
            pltpu.SMEM((1,), f32),
        ],
    )(xyz1, bt)
    return out[0, 0]


# speedup vs baseline: 1.0957x; 1.0491x over previous
import functools

import jax
import jax.numpy as jnp
from jax.experimental import pallas as pl
from jax.experimental.pallas import tpu as pltpu

_WEIGHT = 0.6
_MCHUNK = 2048


def _chamfer_body(a_ref, bt_ref, out_ref, rmin_ref, acc_ref, *, rscale, cscale):
    b = pl.program_id(0)
    mi = pl.program_id(1)
    nb = pl.num_programs(0)
    nm = pl.num_programs(1)
    N = a_ref.shape[1]
    f32 = jnp.float32

    a = a_ref[0]
    bt = bt_ref[0]

    a6 = jnp.concatenate([a, jnp.ones((N, 3), f32)], axis=1)
    bb = jnp.sum(bt * bt, axis=0, keepdims=True)
    hi = bb.astype(jnp.bfloat16).astype(f32)
    r1 = bb - hi
    mid = r1.astype(jnp.bfloat16).astype(f32)
    lo = r1 - mid
    b6 = jnp.concatenate([-2.0 * bt, hi, mid, lo], axis=0)

    ab2 = jax.lax.dot_general(
        a6, b6, (((1,), (0,)), ((), ())),
        preferred_element_type=f32)
    aa = jnp.sum(a * a, axis=1, keepdims=True)

    cmin = jnp.maximum(jnp.min(ab2 + aa, axis=0), 0.0)
    csum = jnp.sum(cmin)

    rmin_chunk = jnp.min(ab2, axis=1, keepdims=True)

    @pl.when(mi == 0)
    def _():
        rmin_ref[...] = rmin_chunk

    @pl.when(mi != 0)
    def _():
        rmin_ref[...] = jnp.minimum(rmin_ref[...], rmin_chunk)

    @pl.when(jnp.logical_and(b == 0, mi == 0))
    def _():
        acc_ref[0] = 0.0

    acc_ref[0] += csum * cscale

    @pl.when(mi == nm - 1)
    def _():
        rsum = jnp.sum(jnp.maximum(rmin_ref[...] + aa, 0.0))
        acc_ref[0] += rsum * rscale

    @pl.when(jnp.logical_and(b == nb - 1, mi == nm - 1))
    def _():
        out_ref[0, 0] = acc_ref[0]


def kernel(xyz1, xyz2):
    B, N, _ = xyz1.shape
    M = xyz2.shape[1]
    f32 = jnp.float32

    bt = jnp.swapaxes(xyz2, 1, 2)

    nm = M // _MCHUNK
    rscale = 0.5 * _WEIGHT / (B * N)
    cscale = 0.5 * _WEIGHT / (B * M)

    out = pl.pallas_call(
        functools.partial(_chamfer_body, rscale=rscale, cscale=cscale),
        grid=(B, nm),
        in_specs=[
            pl.BlockSpec((1, N, 3), lambda b, mi: (b, 0, 0)),
            pl.BlockSpec((1, 3, _MCHUNK), lambda b, mi: (b, 0, mi)),
        ],
        out_specs=pl.BlockSpec(memory_space=pltpu.SMEM),
        out_shape=jax.ShapeDtypeStruct((1, 1), f32),
        scratch_shapes=[
            pltpu.VMEM((N, 1), f32),
            pltpu.SMEM((1,), f32),
        ],
    )(xyz1, bt)
    return out[0, 0]
